# baseline (speedup 1.0000x reference)
import functools

import jax
import jax.numpy as jnp
from jax import lax
from jax.experimental import pallas as pl
from jax.experimental.pallas import tpu as pltpu
from jax.experimental.pallas import tpu_sc as plsc

_SEQ_BLOCK = 1024
_D = 768


def _tc_body(x_ref, p_ref, o_ref):
    o_ref[...] = x_ref[...] + p_ref[...]


def _tc_add(inputs, pos_table, nbatch=None):
    batch, seq, dim = inputs.shape
    if nbatch is None:
        nbatch = batch
    nblk = seq // _SEQ_BLOCK
    return pl.pallas_call(
        _tc_body,
        grid=(nblk,),
        in_specs=[
            pl.BlockSpec((nbatch, _SEQ_BLOCK, dim), lambda i: (0, i, 0)),
            pl.BlockSpec((_SEQ_BLOCK, dim), lambda i: (i, 0)),
        ],
        out_specs=pl.BlockSpec((nbatch, _SEQ_BLOCK, dim), lambda i: (0, i, 0)),
        out_shape=jax.ShapeDtypeStruct((nbatch, seq, dim), inputs.dtype),
    )(inputs, pos_table)


_NC, _NS = 2, 16
_NW = _NC * _NS
_SC_CHUNK = 32


def _sc_body(row_base, total_rows, seq, x_hbm, p_hbm, o_hbm, xbuf, pbuf):
    wid = lax.axis_index("s") * _NC + lax.axis_index("c")
    rows_per_w = total_rows // _NW
    row0 = row_base + wid * rows_per_w
    srow0 = lax.rem(row0, seq)
    ce = _SC_CHUNK * _D

    def chunk_body(i, carry):
        xoff = (row0 + i * _SC_CHUNK) * _D
        ooff = (row0 - row_base + i * _SC_CHUNK) * _D
        poff = (srow0 + i * _SC_CHUNK) * _D
        pltpu.sync_copy(x_hbm.at[pl.ds(xoff, ce)], xbuf)
        pltpu.sync_copy(p_hbm.at[pl.ds(poff, ce)], pbuf)

        def vec_body(k, c2):
            for j in range(8):
                o = (k * 8 + j) * 16
                xbuf[pl.ds(o, 16)] = xbuf[pl.ds(o, 16)] + pbuf[pl.ds(o, 16)]
            return c2

        lax.fori_loop(0, ce // 128, vec_body, 0)
        pltpu.sync_copy(xbuf, o_hbm.at[pl.ds(ooff, ce)])
        return carry

    lax.fori_loop(0, rows_per_w // _SC_CHUNK, chunk_body, 0)


def _sc_add(inputs, pos_table, batch_base=0, nbatch=None):
    batch, seq, dim = inputs.shape
    if nbatch is None:
        nbatch = batch
    row_base = batch_base * seq
    sc_rows = nbatch * seq
    xf = inputs.reshape(batch * seq * dim)
    pf = pos_table.reshape(seq * dim)
    mesh = plsc.VectorSubcoreMesh(core_axis_name="c", subcore_axis_name="s")
    out = pl.kernel(
        functools.partial(_sc_body, row_base, sc_rows, seq),
        mesh=mesh,
        out_type=jax.ShapeDtypeStruct((sc_rows * dim,), jnp.float32),
        scratch_types=[
            pltpu.VMEM((_SC_CHUNK * _D,), jnp.float32),
            pltpu.VMEM((_SC_CHUNK * _D,), jnp.float32),
        ],
    )(xf, pf)
    return out.reshape(nbatch, seq, dim)


_C2 = 32


def _sc2_body(batch, seq, x_hbm, p_hbm, o_hbm,
              xb0, xb1, xb2, pb0, pb1,
              xs0, xs1, xs2, os0, os1, os2, ps0, ps1):
    wid = lax.axis_index("s") * _NC + lax.axis_index("c")
    slab = seq // _NW
    srow0 = wid * slab
    nchunk = slab // _C2
    nunit = nchunk * batch
    xbufs = [xb0, xb1, xb2]
    xsems = [xs0, xs1, xs2]
    osems = [os0, os1, os2]
    pbufs = [pb0, pb1]
    psems = [ps0, ps1]

    def in_src(u):
        i, b = divmod(u, batch)
        return x_hbm.at[b, pl.ds(srow0 + i * _C2, _C2), :]

    def out_dst(u):
        i, b = divmod(u, batch)
        return o_hbm.at[b, pl.ds(srow0 + i * _C2, _C2), :]

    def p_src(i):
        return p_hbm.at[pl.ds(srow0 + i * _C2, _C2), :]

    pltpu.async_copy(p_src(0), pbufs[0], psems[0])
    for u in range(min(2, nunit)):
        pltpu.async_copy(in_src(u), xbufs[u % 3], xsems[u % 3])

    for u in range(nunit):
        i, b = divmod(u, batch)
        if b == 0:
            pltpu.make_async_copy(p_src(i), pbufs[i % 2], psems[i % 2]).wait()
            if i + 1 < nchunk:
                pltpu.async_copy(p_src(i + 1), pbufs[(i + 1) % 2],
                                 psems[(i + 1) % 2])
        xb = xbufs[u % 3]
        pb = pbufs[i % 2]
        pltpu.make_async_copy(in_src(u), xb, xsems[u % 3]).wait()

        def row_body(r, c, xb=xb, pb=pb):
            for jj in range(_D // 16):
                sl = pl.ds(jj * 16, 16)
                xb[r, sl] = xb[r, sl] + pb[r, sl]
            return c

        lax.fori_loop(0, _C2, row_body, 0)
        pltpu.async_copy(xb, out_dst(u), osems[u % 3])
        if u + 2 < nunit:
            if u >= 1:
                pltpu.make_async_copy(xbufs[(u + 2) % 3], out_dst(u - 1),
                                      osems[(u + 2) % 3]).wait()
            pltpu.async_copy(in_src(u + 2), xbufs[(u + 2) % 3],
                             xsems[(u + 2) % 3])

    for u in range(max(0, nunit - 3), nunit):
        pltpu.make_async_copy(xbufs[u % 3], out_dst(u), osems[u % 3]).wait()


def _sc_add_v2(inputs, pos_table):
    batch, seq, dim = inputs.shape
    mesh = plsc.VectorSubcoreMesh(core_axis_name="c", subcore_axis_name="s")
    return pl.kernel(
        functools.partial(_sc2_body, batch, seq),
        mesh=mesh,
        out_type=jax.ShapeDtypeStruct((batch, seq, dim), jnp.float32),
        scratch_types=[
            pltpu.VMEM((_C2, _D), jnp.float32),
            pltpu.VMEM((_C2, _D), jnp.float32),
            pltpu.VMEM((_C2, _D), jnp.float32),
            pltpu.VMEM((_C2, _D), jnp.float32),
            pltpu.VMEM((_C2, _D), jnp.float32),
            pltpu.SemaphoreType.DMA,
            pltpu.SemaphoreType.DMA,
            pltpu.SemaphoreType.DMA,
            pltpu.SemaphoreType.DMA,
            pltpu.SemaphoreType.DMA,
            pltpu.SemaphoreType.DMA,
            pltpu.SemaphoreType.DMA,
            pltpu.SemaphoreType.DMA,
        ],
    )(inputs, pos_table)


def kernel(inputs, pos_table):
    return _sc_add_v2(inputs, pos_table)
